# baseline (speedup 1.0000x reference)
import dataclasses
import functools

import jax
import jax.numpy as jnp
from jax import lax
from jax.experimental import pallas as pl
from jax.experimental.pallas import tpu as pltpu
from jax.experimental.pallas import tpu_sc as plsc

_PH, _PW = 7, 7
_NBINS = _PH * _PW
_LANES = 16
_NEG = float("-inf")
_I32MIN = -2147483648


def _bin_bounds(rois, H, W):
    rois_i = rois.astype(jnp.int32)
    batch_idx = rois_i[:, 0]
    roi_start_w = rois_i[:, 1].astype(jnp.float32)
    roi_start_h = rois_i[:, 2].astype(jnp.float32)
    roi_end_w = rois_i[:, 3].astype(jnp.float32)
    roi_end_h = rois_i[:, 4].astype(jnp.float32)
    roi_height = jnp.maximum(roi_end_h - roi_start_h, 1.0)
    roi_width = jnp.maximum(roi_end_w - roi_start_w, 1.0)
    bin_h = roi_height / float(_PH)
    bin_w = roi_width / float(_PW)
    hs = jnp.arange(_PH, dtype=jnp.float32)
    ws = jnp.arange(_PW, dtype=jnp.float32)
    h_start = jnp.floor(hs[None, :] * bin_h[:, None] + roi_start_h[:, None]).astype(jnp.int32)
    h_end = jnp.ceil((hs[None, :] + 1.0) * bin_h[:, None] + roi_start_h[:, None]).astype(jnp.int32)
    w_start = jnp.floor(ws[None, :] * bin_w[:, None] + roi_start_w[:, None]).astype(jnp.int32)
    w_end = jnp.ceil((ws[None, :] + 1.0) * bin_w[:, None] + roi_start_w[:, None]).astype(jnp.int32)
    h_start = jnp.clip(h_start, 0, H)
    h_end = jnp.clip(h_end, 0, H)
    w_start = jnp.clip(w_start, 0, W)
    w_end = jnp.clip(w_end, 0, W)
    return batch_idx, h_start, h_end, w_start, w_end


def _lane_i32(v, j):
    return jnp.max(jnp.where(lax.iota(jnp.int32, _LANES) == j, v, _I32MIN))


def _make_sc_kernel(B, H, W, C, N, NP):
    HW_C = H * W * C
    n_chunks = C // _LANES
    row = C * _NBINS
    mesh = plsc.VectorSubcoreMesh(
        core_axis_name="c", subcore_axis_name="s", num_cores=2, num_subcores=16
    )
    cp = pltpu.CompilerParams()
    if "needs_layout_passes" in pltpu.CompilerParams.__dataclass_fields__:
        cp = dataclasses.replace(cp, needs_layout_passes=False)

    @functools.partial(
        pl.kernel,
        out_type=jax.ShapeDtypeStruct((N + 2, row), jnp.float32),
        mesh=mesh,
        compiler_params=cp,
        scratch_types=[
            pltpu.VMEM((HW_C,), jnp.float32),
            pltpu.VMEM((NP,), jnp.int32),
            pltpu.VMEM((_LANES,), jnp.int32),
            pltpu.VMEM((row,), jnp.float32),
            pltpu.VMEM((row,), jnp.float32),
            pltpu.VMEM((NP,), jnp.int32),
            pltpu.SemaphoreType.DMA,
            pltpu.SemaphoreType.DMA,
        ],
    )
    def sc_kernel(
        feats_hbm, batch_hbm, packed_hbm, out_hbm,
        feat_v, batch_v, pk_v, out_va, out_vb, list_v, sema, semb,
    ):
        w = lax.axis_index("c") * 16 + lax.axis_index("s")
        pltpu.sync_copy(feats_hbm.at[w], feat_v)
        pltpu.sync_copy(batch_hbm, batch_v)
        lane_iota = lax.iota(jnp.int32, _LANES)
        neg = jnp.full((_LANES,), _NEG, jnp.float32)

        def group(g, cnt):
            bm = batch_v[pl.ds(g * _LANES, _LANES)]
            mask = bm == w
            prefix = plsc.cumsum(jnp.where(mask, 1, 0))
            pos = cnt + prefix - 1
            plsc.store_scatter(list_v, [pos], g * _LANES + lane_iota, mask=mask)
            return cnt + plsc.all_reduce_population_count(mask)

        cnt = lax.fori_loop(0, NP // _LANES, group, jnp.zeros((_LANES,), jnp.int32))
        plsc.store_scatter(
            list_v, [cnt], jnp.broadcast_to(N, (_LANES,)), mask=lane_iota == 0
        )
        total = jnp.max(cnt)
        npairs = (total + 1) // 2

        pltpu.async_copy(out_va, out_hbm.at[N], sema)
        pltpu.async_copy(out_vb, out_hbm.at[N + 1], semb)

        def do_roi(i, out_v, sem):
            grp = (i // _LANES) * _LANES
            lane = i - grp
            v = list_v[pl.ds(grp, _LANES)]
            r = jnp.max(jnp.where(lane_iota == lane, v, _I32MIN))
            pltpu.sync_copy(packed_hbm.at[pl.ds(r * 8, 8)], pk_v.at[pl.ds(0, 8)])
            pk = pk_v[pl.ds(0, _LANES)]
            hsv = pk & 63
            hev = lax.shift_right_logical(pk, 6) & 63
            wsv = lax.shift_right_logical(pk, 12) & 63
            wev = lax.shift_right_logical(pk, 18) & 63
            xs = [_lane_i32(wsv, k) for k in range(_PW)]
            xe = [_lane_i32(wev, k) for k in range(_PW)]
            for hb in range(_PH):
                y0 = _lane_i32(hsv, hb)
                y1 = _lane_i32(hev, hb)
                for wb in range(_PW):
                    x0 = xs[wb]
                    x1 = xe[wb]

                    def yloop(y, accs):
                        def xloop(x, accs):
                            base = (y * W + x) * C
                            return [
                                jnp.maximum(a, feat_v[pl.ds(base + k * _LANES, _LANES)])
                                for k, a in enumerate(accs)
                            ]
                        return lax.fori_loop(x0, x1, xloop, accs)

                    accs = lax.fori_loop(y0, y1, yloop, [neg] * n_chunks)
                    for k in range(n_chunks):
                        val = jnp.where(accs[k] == neg, 0.0, accs[k])
                        idx = (lane_iota + k * _LANES) * _NBINS + (hb * _PW + wb)
                        plsc.store_scatter(out_v, [idx], val)
            pltpu.async_copy(out_v, out_hbm.at[r], sem)

        def per_pair(p, _):
            pltpu.make_async_copy(out_hbm.at[N], out_va, sema).wait()
            do_roi(2 * p, out_va, sema)
            pltpu.make_async_copy(out_hbm.at[N], out_vb, semb).wait()
            do_roi(2 * p + 1, out_vb, semb)
            return 0

        lax.fori_loop(0, npairs, per_pair, 0)

        pltpu.make_async_copy(out_hbm.at[N], out_va, sema).wait()
        pltpu.make_async_copy(out_hbm.at[N], out_vb, semb).wait()

    return sc_kernel


@jax.jit
def kernel(features, rois):
    B, C, H, W = features.shape
    N = rois.shape[0]
    batch_idx, h_start, h_end, w_start, w_end = _bin_bounds(rois, H, W)
    featsT = jnp.transpose(features, (0, 2, 3, 1)).reshape(B, H * W * C)
    NP = ((N + _LANES - 1) // _LANES) * _LANES
    batch_p = jnp.full((NP,), 127, jnp.int32).at[:N].set(batch_idx)
    packed = h_start | (h_end << 6) | (w_start << 12) | (w_end << 18)
    packed = jnp.concatenate([packed, jnp.zeros((N, 1), jnp.int32)], axis=1)
    packed_p = jnp.zeros((NP, 8), jnp.int32).at[:N].set(packed).reshape(-1)
    out = _make_sc_kernel(B, H, W, C, N, NP)(featsT, batch_p, packed_p)
    return out[:N].reshape(N, C, _PH, _PW)


# speedup vs baseline: 1.1308x; 1.1308x over previous
import dataclasses
import functools

import jax
import jax.numpy as jnp
from jax import lax
from jax.experimental import pallas as pl
from jax.experimental.pallas import tpu as pltpu
from jax.experimental.pallas import tpu_sc as plsc

_PH, _PW = 7, 7
_NBINS = _PH * _PW
_LANES = 16
_NEG = float("-inf")
_I32MIN = -2147483648


def _bin_bounds(rois, H, W):
    rois_i = rois.astype(jnp.int32)
    batch_idx = rois_i[:, 0]
    roi_start_w = rois_i[:, 1].astype(jnp.float32)
    roi_start_h = rois_i[:, 2].astype(jnp.float32)
    roi_end_w = rois_i[:, 3].astype(jnp.float32)
    roi_end_h = rois_i[:, 4].astype(jnp.float32)
    roi_height = jnp.maximum(roi_end_h - roi_start_h, 1.0)
    roi_width = jnp.maximum(roi_end_w - roi_start_w, 1.0)
    bin_h = roi_height / float(_PH)
    bin_w = roi_width / float(_PW)
    hs = jnp.arange(_PH, dtype=jnp.float32)
    ws = jnp.arange(_PW, dtype=jnp.float32)
    h_start = jnp.floor(hs[None, :] * bin_h[:, None] + roi_start_h[:, None]).astype(jnp.int32)
    h_end = jnp.ceil((hs[None, :] + 1.0) * bin_h[:, None] + roi_start_h[:, None]).astype(jnp.int32)
    w_start = jnp.floor(ws[None, :] * bin_w[:, None] + roi_start_w[:, None]).astype(jnp.int32)
    w_end = jnp.ceil((ws[None, :] + 1.0) * bin_w[:, None] + roi_start_w[:, None]).astype(jnp.int32)
    h_start = jnp.clip(h_start, 0, H)
    h_end = jnp.clip(h_end, 0, H)
    w_start = jnp.clip(w_start, 0, W)
    w_end = jnp.clip(w_end, 0, W)
    return batch_idx, h_start, h_end, w_start, w_end


def _lane_i32(v, j):
    return jnp.max(jnp.where(lax.iota(jnp.int32, _LANES) == j, v, _I32MIN))


def _make_sc_kernel(B, H, W, C, N, NP):
    HW_C = H * W * C
    n_chunks = C // _LANES
    row = C * _NBINS
    mesh = plsc.VectorSubcoreMesh(
        core_axis_name="c", subcore_axis_name="s", num_cores=2, num_subcores=16
    )
    cp = pltpu.CompilerParams()
    if "needs_layout_passes" in pltpu.CompilerParams.__dataclass_fields__:
        cp = dataclasses.replace(cp, needs_layout_passes=False)

    @functools.partial(
        pl.kernel,
        out_type=jax.ShapeDtypeStruct((N + 2, row), jnp.float32),
        mesh=mesh,
        compiler_params=cp,
        scratch_types=[
            pltpu.VMEM((HW_C,), jnp.float32),
            pltpu.VMEM((NP,), jnp.int32),
            pltpu.VMEM((_LANES,), jnp.int32),
            pltpu.VMEM((row,), jnp.float32),
            pltpu.VMEM((NP,), jnp.int32),
            pltpu.SemaphoreType.DMA,
        ],
    )
    def sc_kernel(
        feats_hbm, batch_hbm, packed_hbm, out_hbm,
        feat_v, batch_v, pk_v, out_v, list_v, sem,
    ):
        w = lax.axis_index("c") * 16 + lax.axis_index("s")
        pltpu.sync_copy(feats_hbm.at[w], feat_v)
        pltpu.sync_copy(batch_hbm, batch_v)
        lane_iota = lax.iota(jnp.int32, _LANES)
        neg = jnp.full((_LANES,), _NEG, jnp.float32)

        def group(g, cnt):
            bm = batch_v[pl.ds(g * _LANES, _LANES)]
            mask = bm == w
            prefix = plsc.cumsum(jnp.where(mask, 1, 0))
            pos = cnt + prefix - 1
            plsc.store_scatter(list_v, [pos], g * _LANES + lane_iota, mask=mask)
            return cnt + plsc.all_reduce_population_count(mask)

        cnt = lax.fori_loop(0, NP // _LANES, group, jnp.zeros((_LANES,), jnp.int32))
        total = jnp.max(cnt)

        def per_roi(i, _):
            grp = (i // _LANES) * _LANES
            lane = i - grp
            v = list_v[pl.ds(grp, _LANES)]
            r = jnp.max(jnp.where(lane_iota == lane, v, _I32MIN))
            pltpu.sync_copy(packed_hbm.at[pl.ds(r * 8, 8)], pk_v.at[pl.ds(0, 8)])
            pk = pk_v[pl.ds(0, _LANES)]
            hsv = pk & 63
            hev = lax.shift_right_logical(pk, 6) & 63
            wsv = lax.shift_right_logical(pk, 12) & 63
            wev = lax.shift_right_logical(pk, 18) & 63
            xs = [_lane_i32(wsv, k) for k in range(_PW)]
            xe = [_lane_i32(wev, k) for k in range(_PW)]
            for hb in range(_PH):
                y0 = _lane_i32(hsv, hb)
                y1 = _lane_i32(hev, hb)
                for wb in range(_PW):
                    x0 = xs[wb]
                    x1 = xe[wb]

                    def yloop(y, accs):
                        def xloop(x, accs):
                            base = (y * W + x) * C
                            return [
                                jnp.maximum(a, feat_v[pl.ds(base + k * _LANES, _LANES)])
                                for k, a in enumerate(accs)
                            ]
                        return lax.fori_loop(x0, x1, xloop, accs)

                    accs = lax.fori_loop(y0, y1, yloop, [neg] * n_chunks)
                    for k in range(n_chunks):
                        val = jnp.where(accs[k] == neg, 0.0, accs[k])
                        idx = (lane_iota + k * _LANES) * _NBINS + (hb * _PW + wb)
                        plsc.store_scatter(out_v, [idx], val)
            pltpu.sync_copy(out_v, out_hbm.at[r])
            return 0

        lax.fori_loop(0, total, per_roi, 0)

    return sc_kernel


@jax.jit
def kernel(features, rois):
    B, C, H, W = features.shape
    N = rois.shape[0]
    batch_idx, h_start, h_end, w_start, w_end = _bin_bounds(rois, H, W)
    featsT = jnp.transpose(features, (0, 2, 3, 1)).reshape(B, H * W * C)
    NP = ((N + _LANES - 1) // _LANES) * _LANES
    batch_p = jnp.full((NP,), 127, jnp.int32).at[:N].set(batch_idx)
    packed = h_start | (h_end << 6) | (w_start << 12) | (w_end << 18)
    packed = jnp.concatenate([packed, jnp.zeros((N, 1), jnp.int32)], axis=1)
    packed_p = jnp.zeros((NP, 8), jnp.int32).at[:N].set(packed).reshape(-1)
    out = _make_sc_kernel(B, H, W, C, N, NP)(featsT, batch_p, packed_p)
    return out[:N].reshape(N, C, _PH, _PW)
